# baseline (speedup 1.0000x reference)
import functools

import jax
import jax.numpy as jnp
from jax import lax
from jax.experimental import pallas as pl
from jax.experimental.pallas import tpu as pltpu
from jax.experimental.pallas import tpu_sc as plsc

EMBED = 128
BATCH = 16384
NFEAT = 4
NC, NS = 2, 16
NW = NC * NS
BPW = BATCH // NW
CH = 128
NBLK = BPW // CH
NT = NFEAT * NBLK
NBUF = 6
TABLE_ROWS = (4, 12, 7, 24)


def _body(x_hbm, t0, t1, t2, t3, out_hbm,
          idx_v, ts0, ts1, ts2, ts3, b0, b1, b2, b3, b4, b5,
          sg0, sg1, sg2, sw0, sw1, sw2, sw3, sw4, sw5):
    sid = lax.axis_index("s")
    wid = sid * NC + lax.axis_index("c")
    base_b = wid * BPW

    tables_sp = (ts0, ts1, ts2, ts3)

    icp = pltpu.async_copy(x_hbm.at[pl.ds(wid * NT, NT)], idx_v, sg0)
    for f, th in enumerate((t0, t1, t2, t3)):
        @pl.when(sid == f)
        def _(th=th, tsp=tables_sp[f]):
            pltpu.sync_copy(th, tsp)
    icp.wait()

    plsc.subcore_barrier()

    bufs = (b0, b1, b2, b3, b4, b5)
    gsems = (sg0, sg1, sg2)
    wsems = (sw0, sw1, sw2, sw3, sw4, sw5)

    def gather(t):
        return pltpu.async_copy(tables_sp[t // NBLK].at[idx_v.at[t]],
                                bufs[t % NBUF], gsems[t % 3])

    def write(t):
        f, q = divmod(t, NBLK)
        return pltpu.async_copy(
            bufs[t % NBUF],
            out_hbm.at[pl.ds(base_b + q * CH, CH),
                       pl.ds(f * EMBED, EMBED)],
            wsems[t % NBUF])

    gcp = [gather(0), gather(1), gather(2)]
    wcp = [None] * NBUF
    for t in range(NT):
        gcp[t % 3].wait()
        wcp[t % NBUF] = write(t)
        n = t + 3
        if n < NT:
            if wcp[n % NBUF] is not None:
                wcp[n % NBUF].wait()
                wcp[n % NBUF] = None
            gcp[n % 3] = gather(n)
    for p in range(NBUF):
        if wcp[p] is not None:
            wcp[p].wait()


_gather = functools.partial(
    pl.kernel,
    out_type=jax.ShapeDtypeStruct((BATCH, NFEAT * EMBED), jnp.float32),
    mesh=plsc.VectorSubcoreMesh(core_axis_name="c", subcore_axis_name="s"),
    scratch_types=[
        pltpu.VMEM((NT, CH), jnp.int32),
        pltpu.VMEM_SHARED((TABLE_ROWS[0], EMBED), jnp.float32),
        pltpu.VMEM_SHARED((TABLE_ROWS[1], EMBED), jnp.float32),
        pltpu.VMEM_SHARED((TABLE_ROWS[2], EMBED), jnp.float32),
        pltpu.VMEM_SHARED((TABLE_ROWS[3], EMBED), jnp.float32),
        pltpu.VMEM((CH, EMBED), jnp.float32),
        pltpu.VMEM((CH, EMBED), jnp.float32),
        pltpu.VMEM((CH, EMBED), jnp.float32),
        pltpu.VMEM((CH, EMBED), jnp.float32),
        pltpu.VMEM((CH, EMBED), jnp.float32),
        pltpu.VMEM((CH, EMBED), jnp.float32),
        pltpu.SemaphoreType.DMA,
        pltpu.SemaphoreType.DMA,
        pltpu.SemaphoreType.DMA,
        pltpu.SemaphoreType.DMA,
        pltpu.SemaphoreType.DMA,
        pltpu.SemaphoreType.DMA,
        pltpu.SemaphoreType.DMA,
        pltpu.SemaphoreType.DMA,
        pltpu.SemaphoreType.DMA,
    ],
)(_body)


@jax.jit
def kernel(x, W_season, W_month, W_day_of_week, W_hour):
    xt = (x[:, 2:6].astype(jnp.int32)
          .T.reshape(NFEAT, NW, BPW)
          .transpose(1, 0, 2)
          .reshape(NW * NT, CH))
    return _gather(xt, W_season, W_month, W_day_of_week, W_hour)
